# baseline (speedup 1.0000x reference)
import functools

import jax
import jax.numpy as jnp
from jax import lax
from jax.experimental import pallas as pl
from jax.experimental.pallas import tpu as pltpu
from jax.experimental.pallas import tpu_sc as plsc

_NUM_FIELDS = 26
_VOCAB = 100000
_EMB = 32
_BATCH = 16384
_NCOMP = _NUM_FIELDS * _EMB
_NW = 32
_CPW = _NCOMP // _NW
_CH = 4096
_NCHK = _BATCH // _CH
_LANES = 16

_mesh = plsc.VectorSubcoreMesh(core_axis_name="c", subcore_axis_name="s")


@functools.partial(
    pl.kernel,
    mesh=_mesh,
    out_type=jax.ShapeDtypeStruct((_NCOMP, _BATCH), jnp.float32),
    scratch_types=[
        pltpu.VMEM((_VOCAB,), jnp.float32),
        pltpu.VMEM((_BATCH,), jnp.int32),
        pltpu.VMEM((2, _CH), jnp.float32),
        pltpu.SemaphoreType.DMA,
        pltpu.SemaphoreType.DMA,
        pltpu.SemaphoreType.DMA,
    ],
    compiler_params=pltpu.CompilerParams(
        use_tc_tiling_on_sc=True, needs_layout_passes=False),
)
def _sc_embed(xt_hbm, tablest_hbm, out_hbm, vec, xrow, ob,
              sem_v, sem_x, sem_st):
    wid = lax.axis_index("s") * 2 + lax.axis_index("c")
    c0 = wid * _CPW

    def component(j, carry):
        cc = c0 + j
        fld = cc // _EMB
        comp = lax.rem(cc, _EMB)
        prev_fld = (cc - 1) // _EMB

        @pl.when(jnp.logical_or(j == 0, fld != prev_fld))
        def _():
            pltpu.async_copy(xt_hbm.at[fld], xrow, sem_x).wait()

        pltpu.async_copy(tablest_hbm.at[fld, comp], vec, sem_v).wait()

        for k in range(_NCHK):
            buf = k % 2
            if k >= 2:
                pltpu.make_async_copy(
                    ob.at[(k - 2) % 2],
                    out_hbm.at[cc, pl.ds((k - 2) * _CH, _CH)],
                    sem_st).wait()

            def gstep(t, carry2):
                idx = xrow[pl.ds(k * _CH + t * _LANES, _LANES)]
                ob[buf, pl.ds(t * _LANES, _LANES)] = plsc.load_gather(
                    vec, [idx])
                return carry2

            lax.fori_loop(0, _CH // _LANES, gstep, 0, unroll=16)
            pltpu.async_copy(
                ob.at[buf], out_hbm.at[cc, pl.ds(k * _CH, _CH)], sem_st)
        for k in (_NCHK - 2, _NCHK - 1):
            pltpu.make_async_copy(
                ob.at[k % 2],
                out_hbm.at[cc, pl.ds(k * _CH, _CH)],
                sem_st).wait()
        return carry

    lax.fori_loop(0, _CPW, component, 0)


def kernel(x_cat, tables):
    out_t = _sc_embed(x_cat.T, tables.transpose(0, 2, 1))
    return out_t.T


# speedup vs baseline: 1.0243x; 1.0243x over previous
import functools

import jax
import jax.numpy as jnp
from jax import lax
from jax.experimental import pallas as pl
from jax.experimental.pallas import tpu as pltpu
from jax.experimental.pallas import tpu_sc as plsc

_NUM_FIELDS = 26
_VOCAB = 100000
_EMB = 32
_BATCH = 16384
_NCOMP = _NUM_FIELDS * _EMB
_NW = 32
_CPW = _NCOMP // _NW
_CH = 2048
_NCHK = _BATCH // _CH
_LANES = 16

_mesh = plsc.VectorSubcoreMesh(core_axis_name="c", subcore_axis_name="s")


@functools.partial(
    pl.kernel,
    mesh=_mesh,
    out_type=jax.ShapeDtypeStruct((_NCOMP, _BATCH), jnp.float32),
    scratch_types=[
        pltpu.VMEM((_VOCAB,), jnp.float32),
        pltpu.VMEM((_BATCH,), jnp.int32),
        pltpu.VMEM((2, _CH), jnp.float32),
        pltpu.SemaphoreType.DMA,
        pltpu.SemaphoreType.DMA,
        pltpu.SemaphoreType.DMA,
    ],
    compiler_params=pltpu.CompilerParams(
        use_tc_tiling_on_sc=True, needs_layout_passes=False),
)
def _sc_embed(xt_hbm, tablest_hbm, out_hbm, vec, xrow, ob,
              sem_v, sem_x, sem_st):
    wid = lax.axis_index("s") * 2 + lax.axis_index("c")
    c0 = wid * _CPW

    pltpu.async_copy(
        tablest_hbm.at[c0 // _EMB, lax.rem(c0, _EMB)], vec, sem_v)

    def component(j, carry):
        cc = c0 + j
        fld = cc // _EMB
        comp = lax.rem(cc, _EMB)
        prev_fld = (cc - 1) // _EMB

        @pl.when(jnp.logical_or(j == 0, fld != prev_fld))
        def _():
            pltpu.async_copy(xt_hbm.at[fld], xrow, sem_x).wait()

        pltpu.make_async_copy(tablest_hbm.at[fld, comp], vec, sem_v).wait()

        for k in range(_NCHK):
            buf = k % 2
            if k >= 2:
                pltpu.make_async_copy(
                    ob.at[(k - 2) % 2],
                    out_hbm.at[cc, pl.ds((k - 2) * _CH, _CH)],
                    sem_st).wait()

            def gstep(t, carry2):
                idx = xrow[pl.ds(k * _CH + t * _LANES, _LANES)]
                ob[buf, pl.ds(t * _LANES, _LANES)] = plsc.load_gather(
                    vec, [idx])
                return carry2

            lax.fori_loop(0, _CH // _LANES, gstep, 0, unroll=8)
            pltpu.async_copy(
                ob.at[buf], out_hbm.at[cc, pl.ds(k * _CH, _CH)], sem_st)
        @pl.when(j + 1 < _CPW)
        def _():
            nc = cc + 1
            pltpu.async_copy(
                tablest_hbm.at[nc // _EMB, lax.rem(nc, _EMB)], vec, sem_v)

        for k in (_NCHK - 2, _NCHK - 1):
            pltpu.make_async_copy(
                ob.at[k % 2],
                out_hbm.at[cc, pl.ds(k * _CH, _CH)],
                sem_st).wait()
        return carry

    lax.fori_loop(0, _CPW, component, 0)


def kernel(x_cat, tables):
    out_t = _sc_embed(x_cat.T, tables.transpose(0, 2, 1))
    return out_t.T


# speedup vs baseline: 1.0289x; 1.0045x over previous
import functools

import jax
import jax.numpy as jnp
from jax import lax
from jax.experimental import pallas as pl
from jax.experimental.pallas import tpu as pltpu
from jax.experimental.pallas import tpu_sc as plsc

_NUM_FIELDS = 26
_VOCAB = 100000
_EMB = 32
_BATCH = 16384
_NCOMP = _NUM_FIELDS * _EMB
_NW = 32
_CPW = _NCOMP // _NW
_CH = 2048
_NCHK = _BATCH // _CH
_LANES = 16

_mesh = plsc.VectorSubcoreMesh(core_axis_name="c", subcore_axis_name="s")


@functools.partial(
    pl.kernel,
    mesh=_mesh,
    out_type=jax.ShapeDtypeStruct((_NCOMP, _BATCH), jnp.float32),
    scratch_types=[
        pltpu.VMEM((_VOCAB,), jnp.float32),
        pltpu.VMEM((_BATCH,), jnp.int32),
        pltpu.VMEM((2, _CH), jnp.float32),
        pltpu.SemaphoreType.DMA,
        pltpu.SemaphoreType.DMA,
        pltpu.SemaphoreType.DMA,
    ],
    compiler_params=pltpu.CompilerParams(
        use_tc_tiling_on_sc=True, needs_layout_passes=False),
)
def _sc_embed(xt_hbm, tablest_hbm, out_hbm, vec, xrow, ob,
              sem_v, sem_x, sem_st):
    wid = lax.axis_index("s") * 2 + lax.axis_index("c")
    c0 = wid * _CPW

    pltpu.async_copy(
        tablest_hbm.at[c0 // _EMB, lax.rem(c0, _EMB)], vec, sem_v)

    def component(j, carry):
        cc = c0 + j
        fld = cc // _EMB
        comp = lax.rem(cc, _EMB)
        prev_fld = (cc - 1) // _EMB

        @pl.when(jnp.logical_or(j == 0, fld != prev_fld))
        def _():
            pltpu.async_copy(xt_hbm.at[fld], xrow, sem_x).wait()

        pltpu.make_async_copy(tablest_hbm.at[fld, comp], vec, sem_v).wait()

        for k in range(_NCHK):
            buf = k % 2
            if k >= 2:
                pltpu.make_async_copy(
                    ob.at[(k - 2) % 2],
                    out_hbm.at[cc, pl.ds((k - 2) * _CH, _CH)],
                    sem_st).wait()

            def gstep(t, carry2):
                idx = xrow[pl.ds(k * _CH + t * _LANES, _LANES)]
                ob[buf, pl.ds(t * _LANES, _LANES)] = plsc.load_gather(
                    vec, [idx])
                return carry2

            lax.fori_loop(0, _CH // _LANES, gstep, 0, unroll=32)
            pltpu.async_copy(
                ob.at[buf], out_hbm.at[cc, pl.ds(k * _CH, _CH)], sem_st)
        @pl.when(j + 1 < _CPW)
        def _():
            nc = cc + 1
            pltpu.async_copy(
                tablest_hbm.at[nc // _EMB, lax.rem(nc, _EMB)], vec, sem_v)

        for k in (_NCHK - 2, _NCHK - 1):
            pltpu.make_async_copy(
                ob.at[k % 2],
                out_hbm.at[cc, pl.ds(k * _CH, _CH)],
                sem_st).wait()
        return carry

    lax.fori_loop(0, _CPW, component, 0)


def kernel(x_cat, tables):
    out_t = _sc_embed(x_cat.T, tables.transpose(0, 2, 1))
    return out_t.T
